# baseline (speedup 1.0000x reference)
import functools

import jax
import jax.numpy as jnp
from jax import lax
from jax.experimental import pallas as pl
from jax.experimental.pallas import tpu as pltpu
from jax.experimental.pallas import tpu_sc as plsc

B = 16384
F = 32
NC, NS, L = 2, 16, 16
NW = NC * NS
BPW = B // NW
W = 128


def _sched(ids_sorted):
    wt = ((ids_sorted // W) * W).reshape(NW, BPW)
    first = jnp.concatenate(
        [jnp.ones((NW, 1), jnp.bool_), wt[:, 1:] != wt[:, :-1]], axis=1)
    slot = jnp.cumsum(first.astype(jnp.int32), axis=1) - 1
    cnt = slot[:, -1] + 1
    rows = jnp.broadcast_to(jnp.arange(NW)[:, None], (NW, BPW))
    cols = jnp.broadcast_to(jnp.arange(BPW)[None, :], (NW, BPW))
    wlist = jnp.zeros((NW, BPW), jnp.int32).at[rows, slot].set(wt)
    rstart = jnp.full((NW, BPW), BPW, jnp.int32).at[rows, slot].min(cols)
    rend = jnp.zeros((NW, BPW), jnp.int32).at[rows, slot].max(cols + 1)
    return (wlist.reshape(-1), rstart.reshape(-1), rend.reshape(-1),
            cnt.astype(jnp.int32))


def _gather_phase(tbl_hbm, wl_v, rs_v, re_v, q_v, cnt, wins, loc, sem):
    iota = lax.iota(jnp.int32, L)

    def fire(s):
        h = plsc.load_gather(wl_v, [jnp.full((L,), s, jnp.int32)])[0]
        par = jnp.asarray(s % 2, jnp.int32)
        return pltpu.async_copy(
            tbl_hbm.at[:, pl.ds(pl.multiple_of(h, W), W)], wins.at[par], sem)

    fire(0)

    def wloop(s, carry):
        par = s % 2
        pltpu.make_async_copy(
            tbl_hbm.at[:, pl.ds(0, W)], wins.at[par], sem).wait()

        @pl.when(s + 1 < cnt)
        def _():
            fire(s + 1)

        rs = plsc.load_gather(rs_v, [jnp.full((L,), s, jnp.int32)])[0]
        re = plsc.load_gather(re_v, [jnp.full((L,), s, jnp.int32)])[0]
        parv = jnp.full((L,), par, jnp.int32)

        def ebody(i, c):
            qv = plsc.load_gather(q_v, [jnp.full((L,), i, jnp.int32)])
            lo = plsc.load_gather(wins, [parv, iota, qv])
            hi = plsc.load_gather(wins, [parv, iota + L, qv])
            loc[pl.ds(i * F, L)] = lo
            loc[pl.ds(i * F + L, L)] = hi
            return c

        lax.fori_loop(rs, re, ebody, 0)
        return carry

    lax.fori_loop(0, cnt, wloop, 0)


def _mfa_body(qu_hbm, qv_hbm, wlu_hbm, rsu_hbm, reu_hbm, cntu_hbm,
              wlv_hbm, rsv_hbm, rev_hbm, cntv_hbm, uft_hbm, ift_hbm,
              us_hbm, vs_hbm,
              wl_v, rs_v, re_v, q_v, cnt_v, wins, loc, sem):
    wid = lax.axis_index("s") * NC + lax.axis_index("c")
    base = wid * BPW

    for (q_hbm, wl_hbm, rs_hbm, re_hbm, cnt_hbm, tbl_hbm, dst_hbm) in (
            (qu_hbm, wlu_hbm, rsu_hbm, reu_hbm, cntu_hbm, uft_hbm, us_hbm),
            (qv_hbm, wlv_hbm, rsv_hbm, rev_hbm, cntv_hbm, ift_hbm, vs_hbm)):
        pltpu.sync_copy(wl_hbm.at[pl.ds(base, BPW)], wl_v)
        pltpu.sync_copy(rs_hbm.at[pl.ds(base, BPW)], rs_v)
        pltpu.sync_copy(re_hbm.at[pl.ds(base, BPW)], re_v)
        pltpu.sync_copy(q_hbm.at[pl.ds(base, BPW)], q_v)
        pltpu.sync_copy(cnt_hbm, cnt_v)
        cnt = plsc.load_gather(cnt_v, [jnp.full((L,), wid, jnp.int32)])[0]
        _gather_phase(tbl_hbm, wl_v, rs_v, re_v, q_v, cnt, wins, loc, sem)
        pltpu.sync_copy(loc, dst_hbm.at[pl.ds(base * F, BPW * F)])


_mfa = functools.partial(
    pl.kernel,
    out_type=(jax.ShapeDtypeStruct((B * F,), jnp.float32),
              jax.ShapeDtypeStruct((B * F,), jnp.float32)),
    mesh=plsc.VectorSubcoreMesh(core_axis_name="c", subcore_axis_name="s",
                                num_cores=NC, num_subcores=NS),
    compiler_params=pltpu.CompilerParams(needs_layout_passes=False),
    scratch_types=[
        pltpu.VMEM((BPW,), jnp.int32),
        pltpu.VMEM((BPW,), jnp.int32),
        pltpu.VMEM((BPW,), jnp.int32),
        pltpu.VMEM((BPW,), jnp.int32),
        pltpu.VMEM((NW,), jnp.int32),
        pltpu.VMEM((2, F, W), jnp.float32),
        pltpu.VMEM((BPW * F,), jnp.float32),
        pltpu.SemaphoreType.DMA,
    ],
)(_mfa_body)


def _mfb_body(ipu_hbm, ipv_hbm, us_hbm, vs_hbm, out_hbm,
              ipu_v, ipv_v, urows, vrows, outv, sem):
    wid = lax.axis_index("s") * NC + lax.axis_index("c")
    base = wid * BPW

    pltpu.sync_copy(ipu_hbm.at[pl.ds(base, BPW)], ipu_v)
    pltpu.sync_copy(ipv_hbm.at[pl.ds(base, BPW)], ipv_v)

    def fire(j, c):
        uvec = ipu_v[pl.ds(j * L, L)]
        vvec = ipv_v[pl.ds(j * L, L)]
        for k in range(L):
            dst = pl.ds((j * L + k) * F, F)
            pltpu.async_copy(us_hbm.at[pl.ds(uvec[k] * F, F)],
                             urows.at[dst], sem)
            pltpu.async_copy(vs_hbm.at[pl.ds(vvec[k] * F, F)],
                             vrows.at[dst], sem)
        return c

    lax.fori_loop(0, BPW // L, fire, 0)
    pltpu.make_async_copy(us_hbm.at[pl.ds(0, BPW * F)], urows, sem).wait()
    pltpu.make_async_copy(us_hbm.at[pl.ds(0, BPW * F)], vrows, sem).wait()

    iota = lax.iota(jnp.int32, L)

    def blk(j, c):
        res = jnp.zeros((L,), jnp.float32)
        for k in range(L):
            off = (j * L + k) * F
            dot = jnp.sum(urows[pl.ds(off, L)] * vrows[pl.ds(off, L)]
                          + urows[pl.ds(off + L, L)] * vrows[pl.ds(off + L, L)],
                          axis=0)
            res = jnp.where(iota == k, dot, res)
        outv[pl.ds(j * L, L)] = res
        return c

    lax.fori_loop(0, BPW // L, blk, 0)
    pltpu.sync_copy(outv, out_hbm.at[pl.ds(base, BPW)])


_mfb = functools.partial(
    pl.kernel,
    out_type=jax.ShapeDtypeStruct((B,), jnp.float32),
    mesh=plsc.VectorSubcoreMesh(core_axis_name="c", subcore_axis_name="s",
                                num_cores=NC, num_subcores=NS),
    compiler_params=pltpu.CompilerParams(needs_layout_passes=False),
    scratch_types=[
        pltpu.VMEM((BPW,), jnp.int32),
        pltpu.VMEM((BPW,), jnp.int32),
        pltpu.VMEM((BPW * F,), jnp.float32),
        pltpu.VMEM((BPW * F,), jnp.float32),
        pltpu.VMEM((BPW,), jnp.float32),
        pltpu.SemaphoreType.DMA,
    ],
)(_mfb_body)


def kernel(user, item, user_factors, item_factors):
    user = user.astype(jnp.int32)
    item = item.astype(jnp.int32)
    pu = jnp.argsort(user)
    pv = jnp.argsort(item)
    su = user[pu]
    sv = item[pv]
    qu = su - (su // W) * W
    qv = sv - (sv // W) * W
    wlu, rsu, reu, cntu = _sched(su)
    wlv, rsv, rev, cntv = _sched(sv)
    arange = jnp.arange(B, dtype=jnp.int32)
    ipu = jnp.zeros((B,), jnp.int32).at[pu].set(arange)
    ipv = jnp.zeros((B,), jnp.int32).at[pv].set(arange)

    us, vs = _mfa(qu, qv, wlu, rsu, reu, cntu, wlv, rsv, rev, cntv,
                  user_factors.T, item_factors.T)
    return _mfb(ipu, ipv, us, vs)


# speedup vs baseline: 2.7510x; 2.7510x over previous
import functools

import jax
import jax.numpy as jnp
from jax import lax
from jax.experimental import pallas as pl
from jax.experimental.pallas import tpu as pltpu
from jax.experimental.pallas import tpu_sc as plsc

B = 16384
F = 32
NC, NS, L = 2, 16, 16
NW = NC * NS
BPW = B // NW
W = 128
K = 8


def _mf_body(user_hbm, item_hbm, uft_hbm, ift_hbm, out_hbm,
             uidx_v, iidx_v, uwins, vwins, outv, sem):
    wid = lax.axis_index("s") * NC + lax.axis_index("c")
    base = wid * BPW

    pltpu.sync_copy(user_hbm.at[pl.ds(base, BPW)], uidx_v)
    pltpu.sync_copy(item_hbm.at[pl.ds(base, BPW)], iidx_v)

    iota = lax.iota(jnp.int32, L)

    def step(j, carry):
        uvec = uidx_v[pl.ds(j * L, L)]
        ivec = iidx_v[pl.ds(j * L, L)]
        uh = (uvec // W) * W
        ih = (ivec // W) * W
        uq = uvec - uh
        iq = ivec - ih
        res = jnp.zeros((L,), jnp.float32)
        for half in range(L // K):
            copies = []
            for k in range(K):
                lane = half * K + k
                copies.append(pltpu.async_copy(
                    uft_hbm.at[:, pl.ds(pl.multiple_of(uh[lane], W), W)],
                    uwins.at[k], sem))
                copies.append(pltpu.async_copy(
                    ift_hbm.at[:, pl.ds(pl.multiple_of(ih[lane], W), W)],
                    vwins.at[k], sem))
            for cp in copies:
                cp.wait()
            for k in range(K):
                lane = half * K + k
                slot = jnp.full((L,), k, jnp.int32)
                uql = jnp.full((L,), uq[lane], jnp.int32)
                iql = jnp.full((L,), iq[lane], jnp.int32)
                ulo = plsc.load_gather(uwins, [slot, iota, uql])
                uhi = plsc.load_gather(uwins, [slot, iota + L, uql])
                vlo = plsc.load_gather(vwins, [slot, iota, iql])
                vhi = plsc.load_gather(vwins, [slot, iota + L, iql])
                dot = jnp.sum(ulo * vlo + uhi * vhi, axis=0)
                res = jnp.where(iota == lane, dot, res)
        outv[pl.ds(j * L, L)] = res
        return carry

    lax.fori_loop(0, BPW // L, step, 0)

    pltpu.sync_copy(outv, out_hbm.at[pl.ds(base, BPW)])


_mf = functools.partial(
    pl.kernel,
    out_type=jax.ShapeDtypeStruct((B,), jnp.float32),
    mesh=plsc.VectorSubcoreMesh(core_axis_name="c", subcore_axis_name="s",
                                num_cores=NC, num_subcores=NS),
    compiler_params=pltpu.CompilerParams(needs_layout_passes=False),
    scratch_types=[
        pltpu.VMEM((BPW,), jnp.int32),
        pltpu.VMEM((BPW,), jnp.int32),
        pltpu.VMEM((K, F, W), jnp.float32),
        pltpu.VMEM((K, F, W), jnp.float32),
        pltpu.VMEM((BPW,), jnp.float32),
        pltpu.SemaphoreType.DMA,
    ],
)(_mf_body)


def kernel(user, item, user_factors, item_factors):
    return _mf(user.astype(jnp.int32), item.astype(jnp.int32),
               user_factors.T, item_factors.T)
